# baseline (speedup 1.0000x reference)
import functools

import jax
import jax.numpy as jnp
from jax import lax
from jax.experimental import pallas as pl
from jax.experimental.pallas import tpu as pltpu
from jax.experimental.pallas import tpu_sc as plsc

_N = 10000
_E = 160000
_DIN = 256
_DOUT = 256
_H = _DOUT // 2
_NSUB = 16
_EPW = _E // _NSUB
_EC = 80
_ECHUNKS = _EPW // _EC
_RC = 80
_RCHUNKS = _N // _RC
_RITER = (_RCHUNKS + _NSUB - 1) // _NSUB


def _matmul_body(x_ref, w_ref, o_ref):
    o_ref[...] = jnp.dot(x_ref[...], w_ref[...],
                         preferred_element_type=jnp.float32)


def _support_table(x, w):
    nb = 10
    bn = _N // nb
    return pl.pallas_call(
        _matmul_body,
        grid=(nb, 2),
        in_specs=[
            pl.BlockSpec((bn, _DIN), lambda i, h: (i, 0)),
            pl.BlockSpec((_DIN, _H), lambda i, h: (0, h)),
        ],
        out_specs=pl.BlockSpec((bn, _H), lambda i, h: (h * nb + i, 0)),
        out_shape=jax.ShapeDtypeStruct((2 * _N, _H), jnp.float32),
    )(x, w)


def _sc_body(sup_hbm, dst_hbm, ew_hbm, bias_hbm, out_hbm,
             acc, src_vm, rows0, rows1, rows2, dstb0, dstb1, dstb2,
             wb0, wb1, wb2, bias_v, gsem0, gsem1, gsem2,
             ssem0, ssem1, ssem2):
    h = lax.axis_index("c")
    s = lax.axis_index("s")
    rows = (rows0, rows1, rows2)
    dstb = (dstb0, dstb1, dstb2)
    wb = (wb0, wb1, wb2)
    gsem = (gsem0, gsem1, gsem2)
    ssem = (ssem0, ssem1, ssem2)

    pltpu.sync_copy(bias_hbm.at[pl.ds(h * _H, _H)], bias_v)

    def _fill_row(i, _):
        for b in range(_H // 16):
            rows0[i, pl.ds(b * 16, 16)] = bias_v[pl.ds(b * 16, 16)]
        return 0
    lax.fori_loop(0, _RC, _fill_row, 0)

    def _init_chunk(k, _):
        t = k * _NSUB + s
        @pl.when(t < _RCHUNKS)
        def _():
            pltpu.sync_copy(rows0, acc.at[pl.ds(t * _RC, _RC)])
        return 0
    lax.fori_loop(0, _RITER, _init_chunk, 0)

    pltpu.sync_copy(dst_hbm.at[pl.ds(s * _EPW, _EPW)], src_vm)

    row_off = h * _N

    def _adjust(i, _):
        src_vm[pl.ds(i * 16, 16)] = src_vm[pl.ds(i * 16, 16)] + row_off
        return 0
    lax.fori_loop(0, _EPW // 16, _adjust, 0, unroll=4)

    plsc.subcore_barrier()

    def _issue(i, b):
        e0 = s * _EPW + i * _EC
        pltpu.async_copy(
            sup_hbm.at[src_vm.at[pl.ds(i * _EC, _EC)]], rows[b], gsem[b])
        pltpu.async_copy(dst_hbm.at[pl.ds(_E + e0, _EC)], dstb[b].at[0],
                         gsem[b])
        pltpu.async_copy(ew_hbm.at[pl.ds(e0, _EC)], wb[b].at[0], gsem[b])

    def _wait_slot(b):
        pltpu.make_async_copy(
            sup_hbm.at[src_vm.at[pl.ds(0, _EC)]], rows[b], gsem[b]).wait()
        pltpu.make_async_copy(
            dst_hbm.at[pl.ds(0, _EC)], dstb[b].at[0], gsem[b]).wait()
        pltpu.make_async_copy(
            ew_hbm.at[pl.ds(0, _EC)], wb[b].at[0], gsem[b]).wait()

    def _wait_scat(b):
        pltpu.make_async_copy(rows[b], acc.at[dstb[b].at[0]], ssem[b]).wait()

    def _visit(i, b, can_issue=True):
        bp = (b + 2) % 3
        _wait_slot(b)

        def _scale(g, _):
            wg = wb[b][0, pl.ds(g * 16, 16)]
            for l in range(16):
                wj = wg[l]
                j = g * 16 + l
                for c in range(_H // 16):
                    rows[b][j, pl.ds(c * 16, 16)] = (
                        rows[b][j, pl.ds(c * 16, 16)] * wj)
            return 0
        lax.fori_loop(0, _EC // 16, _scale, 0, unroll=5)

        pltpu.async_copy(rows[b], acc.at[dstb[b].at[0]], ssem[b], add=True)

        @pl.when(i >= 1)
        def _():
            _wait_scat(bp)
        if can_issue:
            @pl.when(i + 2 < _ECHUNKS)
            def _():
                _issue(i + 2, bp)

    _issue(jnp.int32(0), 0)
    _issue(jnp.int32(1), 1)

    def _outer(g, _):
        _visit(3 * g, 0)
        _visit(3 * g + 1, 1)
        _visit(3 * g + 2, 2)
        return 0
    lax.fori_loop(0, _ECHUNKS // 3, _outer, 0)
    _visit(jnp.int32(_ECHUNKS - 2), 0, can_issue=False)
    _visit(jnp.int32(_ECHUNKS - 1), 1, can_issue=False)
    _wait_scat(1)

    plsc.subcore_barrier()

    def _drain(k, _):
        t = k * _NSUB + s
        @pl.when(t < _RCHUNKS)
        def _():
            r0 = t * _RC
            pltpu.sync_copy(acc.at[pl.ds(r0, _RC)], rows0)
            pltpu.sync_copy(rows0,
                            out_hbm.at[pl.ds(r0, _RC), pl.ds(h * _H, _H)])
        return 0
    lax.fori_loop(0, _RITER, _drain, 0)


_sc_agg = functools.partial(
    pl.kernel,
    out_type=jax.ShapeDtypeStruct((_N, _DOUT), jnp.float32),
    mesh=plsc.VectorSubcoreMesh(core_axis_name="c", subcore_axis_name="s"),
    scratch_types=[
        pltpu.VMEM_SHARED((_N, _H), jnp.float32),
        pltpu.VMEM((_EPW,), jnp.int32),
        pltpu.VMEM((_EC, _H), jnp.float32),
        pltpu.VMEM((_EC, _H), jnp.float32),
        pltpu.VMEM((_EC, _H), jnp.float32),
        pltpu.VMEM((1, _EC), jnp.int32),
        pltpu.VMEM((1, _EC), jnp.int32),
        pltpu.VMEM((1, _EC), jnp.int32),
        pltpu.VMEM((1, _EC), jnp.float32),
        pltpu.VMEM((1, _EC), jnp.float32),
        pltpu.VMEM((1, _EC), jnp.float32),
        pltpu.VMEM((_H,), jnp.float32),
        pltpu.SemaphoreType.DMA,
        pltpu.SemaphoreType.DMA,
        pltpu.SemaphoreType.DMA,
        pltpu.SemaphoreType.DMA,
        pltpu.SemaphoreType.DMA,
        pltpu.SemaphoreType.DMA,
    ],
)(_sc_body)


def kernel(input_feature, edge_index, edge_weight, kernel, bias):
    support = _support_table(input_feature, kernel)
    edges = edge_index.reshape(2 * _E)
    return _sc_agg(support, edges, edge_weight, bias)


# speedup vs baseline: 1.2474x; 1.2474x over previous
import functools

import jax
import jax.numpy as jnp
from jax import lax
from jax.experimental import pallas as pl
from jax.experimental.pallas import tpu as pltpu
from jax.experimental.pallas import tpu_sc as plsc

_N = 10000
_E = 160000
_DIN = 256
_DOUT = 256
_H = _DOUT // 2
_NSUB = 16
_EPW = _E // _NSUB
_EC = 80
_ECHUNKS = _EPW // _EC
_RC = 80
_RCHUNKS = _N // _RC
_RITER = (_RCHUNKS + _NSUB - 1) // _NSUB


def _matmul_body(x_ref, w_ref, o_ref):
    o_ref[...] = jnp.dot(x_ref[...], w_ref[...],
                         preferred_element_type=jnp.float32)


def _support_table(x, w):
    nb = 10
    bn = _N // nb
    return pl.pallas_call(
        _matmul_body,
        grid=(nb, 2),
        in_specs=[
            pl.BlockSpec((bn, _DIN), lambda i, h: (i, 0)),
            pl.BlockSpec((_DIN, _H), lambda i, h: (0, h)),
        ],
        out_specs=pl.BlockSpec((bn, _H), lambda i, h: (h * nb + i, 0)),
        out_shape=jax.ShapeDtypeStruct((2 * _N, _H), jnp.float32),
    )(x, w)


def _sc_body(sup_hbm, dst_hbm, ew_hbm, bias_hbm, out_hbm,
             acc, src_vm, rows0, rows1, rows2, dstb0, dstb1, dstb2,
             wb0, wb1, wb2, bias_v, gsem0, gsem1, gsem2,
             ssem0, ssem1, ssem2):
    h = lax.axis_index("c")
    s = lax.axis_index("s")
    rows = (rows0, rows1, rows2)
    dstb = (dstb0, dstb1, dstb2)
    wb = (wb0, wb1, wb2)
    gsem = (gsem0, gsem1, gsem2)
    ssem = (ssem0, ssem1, ssem2)

    pltpu.sync_copy(bias_hbm.at[pl.ds(h * _H, _H)], bias_v)

    def _fill_row(i, _):
        for b in range(_H // 16):
            rows0[i, pl.ds(b * 16, 16)] = bias_v[pl.ds(b * 16, 16)]
        return 0
    lax.fori_loop(0, _RC, _fill_row, 0)

    def _init_chunk(k, _):
        t = k * _NSUB + s
        @pl.when(t < _RCHUNKS)
        def _():
            pltpu.sync_copy(rows0, acc.at[pl.ds(t * _RC, _RC)])
        return 0
    lax.fori_loop(0, _RITER, _init_chunk, 0)

    pltpu.sync_copy(dst_hbm.at[pl.ds(s * _EPW, _EPW)], src_vm)

    row_off = h * _N

    def _adjust(i, _):
        src_vm[pl.ds(i * 16, 16)] = src_vm[pl.ds(i * 16, 16)] + row_off
        return 0
    lax.fori_loop(0, _EPW // 16, _adjust, 0)

    plsc.subcore_barrier()

    def _issue(i, b):
        e0 = s * _EPW + i * _EC
        pltpu.async_copy(
            sup_hbm.at[src_vm.at[pl.ds(i * _EC, _EC)]], rows[b], gsem[b])
        pltpu.async_copy(dst_hbm.at[pl.ds(_E + e0, _EC)], dstb[b].at[0],
                         gsem[b])
        pltpu.async_copy(ew_hbm.at[pl.ds(e0, _EC)], wb[b].at[0], gsem[b])

    def _wait_slot(b):
        pltpu.make_async_copy(
            sup_hbm.at[src_vm.at[pl.ds(0, _EC)]], rows[b], gsem[b]).wait()
        pltpu.make_async_copy(
            dst_hbm.at[pl.ds(0, _EC)], dstb[b].at[0], gsem[b]).wait()
        pltpu.make_async_copy(
            ew_hbm.at[pl.ds(0, _EC)], wb[b].at[0], gsem[b]).wait()

    def _wait_scat(b):
        pltpu.make_async_copy(rows[b], acc.at[dstb[b].at[0]], ssem[b]).wait()

    def _visit(i, b, can_issue=True):
        bp = (b + 2) % 3
        _wait_slot(b)

        def _scale(g, _):
            wg = wb[b][0, pl.ds(g * 16, 16)]
            for l in range(16):
                wj = wg[l]
                j = g * 16 + l
                for c in range(_H // 16):
                    rows[b][j, pl.ds(c * 16, 16)] = (
                        rows[b][j, pl.ds(c * 16, 16)] * wj)
            return 0
        lax.fori_loop(0, _EC // 16, _scale, 0)

        pltpu.async_copy(rows[b], acc.at[dstb[b].at[0]], ssem[b], add=True)

        @pl.when(i >= 1)
        def _():
            _wait_scat(bp)
        if can_issue:
            @pl.when(i + 2 < _ECHUNKS)
            def _():
                _issue(i + 2, bp)

    _issue(jnp.int32(0), 0)
    _issue(jnp.int32(1), 1)

    def _outer(g, _):
        _visit(3 * g, 0)
        _visit(3 * g + 1, 1)
        _visit(3 * g + 2, 2)
        return 0
    lax.fori_loop(0, _ECHUNKS // 3, _outer, 0)
    _visit(jnp.int32(_ECHUNKS - 2), 0, can_issue=False)
    _visit(jnp.int32(_ECHUNKS - 1), 1, can_issue=False)
    _wait_scat(1)

    plsc.subcore_barrier()

    def _drain(k, _):
        t = k * _NSUB + s
        @pl.when(t < _RCHUNKS)
        def _():
            r0 = t * _RC
            pltpu.sync_copy(acc.at[pl.ds(r0, _RC)], rows0)
            pltpu.sync_copy(rows0,
                            out_hbm.at[pl.ds(r0, _RC), pl.ds(h * _H, _H)])
        return 0
    lax.fori_loop(0, _RITER, _drain, 0)


_sc_agg = functools.partial(
    pl.kernel,
    out_type=jax.ShapeDtypeStruct((_N, _DOUT), jnp.float32),
    mesh=plsc.VectorSubcoreMesh(core_axis_name="c", subcore_axis_name="s"),
    scratch_types=[
        pltpu.VMEM_SHARED((_N, _H), jnp.float32),
        pltpu.VMEM((_EPW,), jnp.int32),
        pltpu.VMEM((_EC, _H), jnp.float32),
        pltpu.VMEM((_EC, _H), jnp.float32),
        pltpu.VMEM((_EC, _H), jnp.float32),
        pltpu.VMEM((1, _EC), jnp.int32),
        pltpu.VMEM((1, _EC), jnp.int32),
        pltpu.VMEM((1, _EC), jnp.int32),
        pltpu.VMEM((1, _EC), jnp.float32),
        pltpu.VMEM((1, _EC), jnp.float32),
        pltpu.VMEM((1, _EC), jnp.float32),
        pltpu.VMEM((_H,), jnp.float32),
        pltpu.SemaphoreType.DMA,
        pltpu.SemaphoreType.DMA,
        pltpu.SemaphoreType.DMA,
        pltpu.SemaphoreType.DMA,
        pltpu.SemaphoreType.DMA,
        pltpu.SemaphoreType.DMA,
    ],
)(_sc_body)


def kernel(input_feature, edge_index, edge_weight, kernel, bias):
    support = _support_table(input_feature, kernel)
    edges = edge_index.reshape(2 * _E)
    return _sc_agg(support, edges, edge_weight, bias)
